# baseline (speedup 1.0000x reference)
import functools

import jax
import jax.numpy as jnp
from jax import lax
from jax.experimental import pallas as pl
from jax.experimental.pallas import tpu as pltpu
from jax.experimental.pallas import tpu_sc as plsc

VOCAB = 10000
D = 300
DP = 304
B = 4096 * 50
NC, NS = 2, 16
NW = NC * NS
BPW = B // NW
CHUNK = 64
NCHUNK = BPW // CHUNK
NVREG = DP // 16
FLAT = CHUNK * D


def _sc_gather(table, idx2d):
  mesh = plsc.VectorSubcoreMesh(core_axis_name="c", subcore_axis_name="s")

  @functools.partial(
      pl.kernel,
      mesh=mesh,
      out_type=jax.ShapeDtypeStruct((B * D,), jnp.float32),
      scratch_types=[
          pltpu.VMEM((NCHUNK, CHUNK), jnp.int32),
          pltpu.VMEM((2, CHUNK, DP), jnp.float32),
          pltpu.VMEM((2, FLAT + 16), jnp.float32),
          pltpu.SemaphoreType.DMA,
          pltpu.SemaphoreType.DMA,
          pltpu.SemaphoreType.DMA,
          pltpu.SemaphoreType.DMA,
      ],
      compiler_params=pltpu.CompilerParams(use_tc_tiling_on_sc=False),
  )
  def k(table_hbm, idx_hbm, out_hbm, idx_v, rows_v, packed_v, g0, g1, s0, s1):
    gsem = (g0, g1)
    ssem = (s0, s1)
    wid = lax.axis_index("s") * NC + lax.axis_index("c")
    fbase = wid * BPW * D
    pltpu.sync_copy(idx_hbm.at[pl.ds(wid * NCHUNK, NCHUNK)], idx_v)

    def gather(c, b):
      return pltpu.make_async_copy(
          table_hbm.at[idx_v.at[c]], rows_v.at[b], gsem[b]
      )

    def write(c, b):
      return pltpu.make_async_copy(
          packed_v.at[b, pl.ds(0, FLAT)],
          out_hbm.at[pl.ds(fbase + c * FLAT, FLAT)],
          ssem[b],
      )

    for b in range(2):
      gather(b, b).start()

    def body(p, carry):
      for b in range(2):
        c = 2 * p + b
        gather(c, b).wait()

        @pl.when(c >= 2)
        def _():
          write(c - 2, b).wait()

        def repack(r, cr):
          src = rows_v.at[b]
          dst = packed_v.at[b]
          for kv in range(NVREG):
            dst[pl.ds(r * D + 16 * kv, 16)] = src[r, pl.ds(16 * kv, 16)]
          return cr

        lax.fori_loop(0, CHUNK, repack, 0)
        write(c, b).start()

        @pl.when(c + 2 < NCHUNK)
        def _():
          gather(c + 2, b).start()
      return carry

    lax.fori_loop(0, NCHUNK // 2, body, 0)
    for b in range(2):
      write(NCHUNK - 2 + b, b).wait()

  return k(table, idx2d)


def kernel(data, ivectors):
  table = jnp.pad(ivectors, ((0, 0), (0, DP - D)))
  idx2d = data.reshape(B // CHUNK, CHUNK).astype(jnp.int32)
  out = _sc_gather(table, idx2d)
  return out.reshape(data.shape[0], data.shape[1], D)


# speedup vs baseline: 1.4721x; 1.4721x over previous
import functools

import jax
import jax.numpy as jnp
from jax import lax
from jax.experimental import pallas as pl
from jax.experimental.pallas import tpu as pltpu
from jax.experimental.pallas import tpu_sc as plsc

VOCAB = 10000
D = 300
DP = 304
NB = 4096
T = 50
NC, NS = 2, 16
NW = NC * NS
SPW = NB // NW


def _sc_gather(table, idx):
  mesh = plsc.VectorSubcoreMesh(core_axis_name="c", subcore_axis_name="s")

  @functools.partial(
      pl.kernel,
      mesh=mesh,
      out_type=jax.ShapeDtypeStruct((NB, T, D), jnp.float32),
      scratch_types=[
          pltpu.VMEM((SPW, T), jnp.int32),
          pltpu.VMEM((2, T, DP), jnp.float32),
          pltpu.VMEM((2, T, D), jnp.float32),
          pltpu.SemaphoreType.DMA,
          pltpu.SemaphoreType.DMA,
          pltpu.SemaphoreType.DMA,
          pltpu.SemaphoreType.DMA,
      ],
      compiler_params=pltpu.CompilerParams(
          use_tc_tiling_on_sc=False, needs_layout_passes=False
      ),
  )
  def k(table_hbm, idx_hbm, out_hbm, idx_v, rows_v, slab_v, g0, g1, s0, s1):
    gsem = (g0, g1)
    ssem = (s0, s1)
    wid = lax.axis_index("s") * NC + lax.axis_index("c")
    sbase = wid * SPW
    pltpu.sync_copy(idx_hbm.at[pl.ds(sbase, SPW)], idx_v)

    iota = jax.lax.iota(jnp.int32, 16)
    col4 = 296 + (iota & 3)
    row4 = iota >> 2

    def gather(c, b):
      return pltpu.make_async_copy(
          table_hbm.at[idx_v.at[c]], rows_v.at[b], gsem[b]
      )

    def write(c, b):
      return pltpu.make_async_copy(
          slab_v.at[b], out_hbm.at[sbase + c], ssem[b]
      )

    for b in range(2):
      gather(b, b).start()

    def body(p, carry):
      for b in range(2):
        c = 2 * p + b
        gather(c, b).wait()

        @pl.when(c >= 2)
        def _():
          write(c - 2, b).wait()

        src = rows_v.at[b]
        dst = slab_v.at[b]
        for t in range(T):
          for kv in range(18):
            dst[t, pl.ds(16 * kv, 16)] = src[t, pl.ds(16 * kv, 16)]
          dst[t, pl.ds(280, 16)] = src[t, pl.ds(280, 16)]
        for g in range(13):
          rr = row4 + 4 * g
          mask = iota < 8 if g == 12 else None
          vals = plsc.load_gather(src, [rr, col4], mask=mask)
          plsc.store_scatter(dst, [rr, col4], vals, mask=mask)

        write(c, b).start()

        @pl.when(c + 2 < SPW)
        def _():
          gather(c + 2, b).start()
      return carry

    lax.fori_loop(0, SPW // 2, body, 0)
    for b in range(2):
      write(SPW - 2 + b, b).wait()

  return k(table, idx)


def kernel(data, ivectors):
  table = jnp.pad(ivectors, ((0, 0), (0, DP - D)))
  return _sc_gather(table, data.astype(jnp.int32))
